# baseline (speedup 1.0000x reference)
import functools

import jax
import jax.numpy as jnp
from jax import lax
from jax.experimental import pallas as pl
from jax.experimental.pallas import tpu as pltpu
from jax.experimental.pallas import tpu_sc as plsc

B = 16384
D = 32
L = 16
NC = 2
NS = 16
NW = NC * NS
BPW = B // NW
C = 256
NCHUNK = BPW // C
GPC = C // L

_mesh = plsc.VectorSubcoreMesh(core_axis_name="c", subcore_axis_name="s")


@functools.partial(
    pl.kernel,
    mesh=_mesh,
    out_type=jax.ShapeDtypeStruct((B,), jnp.float32),
    compiler_params=pltpu.CompilerParams(needs_layout_passes=False),
    scratch_types=[
        pltpu.VMEM((BPW,), jnp.int32),
        pltpu.VMEM((BPW,), jnp.int32),
        pltpu.VMEM((C,), jnp.int32),
        pltpu.VMEM((C,), jnp.int32),
        pltpu.VMEM((C,), jnp.int32),
        pltpu.VMEM((C,), jnp.int32),
        pltpu.VMEM((C, 128), jnp.float32),
        pltpu.VMEM((C, 128), jnp.float32),
        pltpu.VMEM((BPW,), jnp.float32),
        pltpu.VMEM((BPW,), jnp.float32),
        pltpu.VMEM((BPW,), jnp.float32),
        pltpu.SemaphoreType.DMA,
        pltpu.SemaphoreType.DMA,
        pltpu.SemaphoreType.DMA,
        pltpu.SemaphoreType.DMA,
    ],
)
def _mf_sc(users_hbm, items_hbm, ut_hbm, it_hbm, ub_hbm, ib_hbm, out_hbm,
           uidx, iidx, usup0, usup1, isup0, isup1, urows, irows, ubv, ibv,
           res, sem_u, sem_i, sem_ub, sem_ib):
    usups = (usup0, usup1)
    isups = (isup0, isup1)
    wid = lax.axis_index("s") * NC + lax.axis_index("c")
    base = wid * BPW

    pltpu.sync_copy(users_hbm.at[pl.ds(base, BPW)], uidx)
    pltpu.sync_copy(items_hbm.at[pl.ds(base, BPW)], iidx)

    cub = pltpu.async_copy(ub_hbm.at[uidx], ubv, sem_ub)
    cib = pltpu.async_copy(ib_hbm.at[iidx], ibv, sem_ib)

    for s in range(BPW // L):
        u = uidx[pl.ds(s * L, L)]
        v = iidx[pl.ds(s * L, L)]
        usups[s * L // C][pl.ds((s * L) % C, L)] = u >> 2
        isups[s * L // C][pl.ds((s * L) % C, L)] = v >> 2

    cub.wait()
    cib.wait()

    iota = lax.iota(jnp.int32, L)

    for c in range(NCHUNK):
        cu = pltpu.async_copy(ut_hbm.at[usups[c]], urows, sem_u)
        ci = pltpu.async_copy(it_hbm.at[isups[c]], irows, sem_i)
        cu.wait()
        ci.wait()

        def group_body(g, carry, c=c):
            off = c * C + g * L
            rows = iota + g * L
            uvec = uidx[pl.ds(off, L)]
            ivec = iidx[pl.ds(off, L)]
            ucol = (uvec & 3) << 5
            icol = (ivec & 3) << 5
            acc = ubv[pl.ds(off, L)] + ibv[pl.ds(off, L)]
            for j in range(D):
                u = plsc.load_gather(urows, [rows, ucol + j])
                v = plsc.load_gather(irows, [rows, icol + j])
                acc = acc + u * v
            res[pl.ds(off, L)] = acc
            return carry

        lax.fori_loop(0, GPC, group_body, 0)

    pltpu.sync_copy(res, out_hbm.at[pl.ds(base, BPW)])


def kernel(users, items, user_table, item_table, user_bias, item_bias):
    ut = user_table.reshape(-1, 128)
    it = item_table.reshape(-1, 128)
    return _mf_sc(users, items, ut, it, user_bias, item_bias)


# speedup vs baseline: 3.4698x; 3.4698x over previous
import functools

import jax
import jax.numpy as jnp
from jax import lax
from jax.experimental import pallas as pl
from jax.experimental.pallas import tpu as pltpu
from jax.experimental.pallas import tpu_sc as plsc

B = 16384
D = 32
L = 16
NC = 2
NS = 16
NW = NC * NS
BPW = B // NW
NB = 2
NPAIR = BPW // (2 * NB)
GROUPS = BPW // L

_mesh = plsc.VectorSubcoreMesh(core_axis_name="c", subcore_axis_name="s")


@functools.partial(
    pl.kernel,
    mesh=_mesh,
    out_type=jax.ShapeDtypeStruct((B,), jnp.float32),
    compiler_params=pltpu.CompilerParams(needs_layout_passes=False),
    scratch_types=[
        pltpu.VMEM((BPW,), jnp.int32),
        pltpu.VMEM((BPW,), jnp.int32),
        pltpu.VMEM((2, NB, D, 128), jnp.float32),
        pltpu.VMEM((2, NB, D, 128), jnp.float32),
        pltpu.VMEM((BPW, L), jnp.float32),
        pltpu.VMEM((BPW,), jnp.float32),
        pltpu.VMEM((BPW,), jnp.float32),
        pltpu.VMEM((BPW,), jnp.float32),
        pltpu.SemaphoreType.DMA,
        pltpu.SemaphoreType.DMA,
        pltpu.SemaphoreType.DMA,
        pltpu.SemaphoreType.DMA,
        pltpu.SemaphoreType.DMA,
        pltpu.SemaphoreType.DMA,
    ],
)
def _mf_sc(users_hbm, items_hbm, utt_hbm, itt_hbm, ub_hbm, ib_hbm, out_hbm,
           uidx, iidx, ublk, iblk, psum, ubv, ibv,
           res, sem_ua, sem_ia, sem_ub_blk, sem_ib_blk, sem_ubias, sem_ibias):
    wid = lax.axis_index("s") * NC + lax.axis_index("c")
    base = wid * BPW

    pltpu.sync_copy(users_hbm.at[pl.ds(base, BPW)], uidx)
    pltpu.sync_copy(items_hbm.at[pl.ds(base, BPW)], iidx)

    pltpu.async_copy(ub_hbm.at[uidx], ubv, sem_ubias)
    pltpu.async_copy(ib_hbm.at[iidx], ibv, sem_ibias)

    iota = lax.iota(jnp.int32, L)
    c_lo = iota
    c_hi = iota + L

    def idx_scalar(ref, e):
        vec = ref[pl.ds((e >> 4) << 4, L)]
        sel = iota == (e & 15)
        return jnp.sum(jnp.where(sel, vec, 0))

    def fire_batch(e0, slot, sem_u, sem_i):
        for b in range(NB):
            u = idx_scalar(uidx, e0 + b)
            v = idx_scalar(iidx, e0 + b)
            uoff = pl.multiple_of((u >> 7) << 7, 128)
            voff = pl.multiple_of((v >> 7) << 7, 128)
            pltpu.async_copy(
                utt_hbm.at[:, pl.ds(uoff, 128)], ublk.at[slot, b], sem_u
            )
            pltpu.async_copy(
                itt_hbm.at[:, pl.ds(voff, 128)], iblk.at[slot, b], sem_i
            )

    def drain(slot, sem_u, sem_i):
        for b in range(NB):
            pltpu.make_async_copy(
                utt_hbm.at[:, pl.ds(0, 128)], ublk.at[slot, b], sem_u
            ).wait()
            pltpu.make_async_copy(
                itt_hbm.at[:, pl.ds(0, 128)], iblk.at[slot, b], sem_i
            ).wait()

    def extract_batch(e0, slot):
        for b in range(NB):
            e = e0 + b
            ulane = jnp.full((L,), idx_scalar(uidx, e) & 127, jnp.int32)
            vlane = jnp.full((L,), idx_scalar(iidx, e) & 127, jnp.int32)
            bb = jnp.full((L,), b, jnp.int32)
            ss = jnp.full((L,), slot, jnp.int32)
            u0 = plsc.load_gather(ublk, [ss, bb, c_lo, ulane])
            u1 = plsc.load_gather(ublk, [ss, bb, c_hi, ulane])
            v0 = plsc.load_gather(iblk, [ss, bb, c_lo, vlane])
            v1 = plsc.load_gather(iblk, [ss, bb, c_hi, vlane])
            psum[e, pl.ds(0, L)] = u0 * v0 + u1 * v1

    fire_batch(0, 0, sem_ua, sem_ia)

    def pipe_body(t, carry):
        ea = 2 * NB * t
        eb = ea + NB
        fire_batch(eb, 1, sem_ub_blk, sem_ib_blk)
        drain(0, sem_ua, sem_ia)
        extract_batch(ea, 0)

        @pl.when(t < NPAIR - 1)
        def _():
            fire_batch(eb + NB, 0, sem_ua, sem_ia)

        drain(1, sem_ub_blk, sem_ib_blk)
        extract_batch(eb, 1)
        return carry

    lax.fori_loop(0, NPAIR, pipe_body, 0)

    pltpu.make_async_copy(ub_hbm.at[pl.ds(0, BPW)], ubv, sem_ubias).wait()
    pltpu.make_async_copy(ib_hbm.at[pl.ds(0, BPW)], ibv, sem_ibias).wait()

    def group_body(g, carry):
        off = g * L
        rows = iota + off
        acc = ubv[pl.ds(off, L)] + ibv[pl.ds(off, L)]
        for j in range(L):
            col = jnp.full((L,), j, jnp.int32)
            acc = acc + plsc.load_gather(psum, [rows, col])
        res[pl.ds(off, L)] = acc
        return carry

    lax.fori_loop(0, GROUPS, group_body, 0)

    pltpu.sync_copy(res, out_hbm.at[pl.ds(base, BPW)])


def kernel(users, items, user_table, item_table, user_bias, item_bias):
    return _mf_sc(users, items, user_table.T, item_table.T,
                  user_bias, item_bias)


# speedup vs baseline: 3.4857x; 1.0046x over previous
import functools

import jax
import jax.numpy as jnp
from jax import lax
from jax.experimental import pallas as pl
from jax.experimental.pallas import tpu as pltpu
from jax.experimental.pallas import tpu_sc as plsc

B = 16384
D = 32
L = 16
NC = 2
NS = 16
NW = NC * NS
BPW = B // NW
NB = 2
ROUNDS = BPW // L
BPR = L // NB

_mesh = plsc.VectorSubcoreMesh(core_axis_name="c", subcore_axis_name="s")


@functools.partial(
    pl.kernel,
    mesh=_mesh,
    out_type=jax.ShapeDtypeStruct((B,), jnp.float32),
    compiler_params=pltpu.CompilerParams(needs_layout_passes=False),
    scratch_types=[
        pltpu.VMEM((BPW,), jnp.int32),
        pltpu.VMEM((BPW,), jnp.int32),
        pltpu.VMEM((2, NB, D, 128), jnp.float32),
        pltpu.VMEM((2, NB, D, 128), jnp.float32),
        pltpu.VMEM((BPW, L), jnp.float32),
        pltpu.VMEM((BPW,), jnp.float32),
        pltpu.VMEM((BPW,), jnp.float32),
        pltpu.VMEM((BPW,), jnp.float32),
        pltpu.SemaphoreType.DMA,
        pltpu.SemaphoreType.DMA,
        pltpu.SemaphoreType.DMA,
        pltpu.SemaphoreType.DMA,
        pltpu.SemaphoreType.DMA,
        pltpu.SemaphoreType.DMA,
    ],
)
def _mf_sc(users_hbm, items_hbm, utt_hbm, itt_hbm, ub_hbm, ib_hbm, out_hbm,
           uidx, iidx, ublk, iblk, psum, ubv, ibv, res,
           sem_ua, sem_ia, sem_ub_blk, sem_ib_blk, sem_ubias, sem_ibias):
    wid = lax.axis_index("s") * NC + lax.axis_index("c")
    base = wid * BPW

    pltpu.sync_copy(users_hbm.at[pl.ds(base, BPW)], uidx)
    pltpu.sync_copy(items_hbm.at[pl.ds(base, BPW)], iidx)

    pltpu.async_copy(ub_hbm.at[uidx], ubv, sem_ubias)
    pltpu.async_copy(ib_hbm.at[iidx], ibv, sem_ibias)

    iota = lax.iota(jnp.int32, L)
    c_lo = iota
    c_hi = iota + L
    sems = ((sem_ua, sem_ia), (sem_ub_blk, sem_ib_blk))

    def load_round_vecs(t):
        uvec = uidx[pl.ds(t * L, L)]
        ivec = iidx[pl.ds(t * L, L)]
        return ((uvec >> 7) << 7, uvec & 127, (ivec >> 7) << 7, ivec & 127)

    def fire_batch(vecs, k, slot):
        uoffv, _, ioffv, _ = vecs
        sem_u, sem_i = sems[slot]
        for b in range(NB):
            j = k * NB + b
            uoff = pl.multiple_of(uoffv[j], 128)
            ioff = pl.multiple_of(ioffv[j], 128)
            pltpu.async_copy(
                utt_hbm.at[:, pl.ds(uoff, 128)], ublk.at[slot, b], sem_u
            )
            pltpu.async_copy(
                itt_hbm.at[:, pl.ds(ioff, 128)], iblk.at[slot, b], sem_i
            )

    def drain(slot):
        sem_u, sem_i = sems[slot]
        for b in range(NB):
            pltpu.make_async_copy(
                utt_hbm.at[:, pl.ds(0, 128)], ublk.at[slot, b], sem_u
            ).wait()
            pltpu.make_async_copy(
                itt_hbm.at[:, pl.ds(0, 128)], iblk.at[slot, b], sem_i
            ).wait()

    def extract_batch(e0, vecs, k, slot):
        _, ulanev, _, ilanev = vecs
        ss = jnp.full((L,), slot, jnp.int32)
        for b in range(NB):
            j = k * NB + b
            ulane = jnp.full((L,), ulanev[j], jnp.int32)
            vlane = jnp.full((L,), ilanev[j], jnp.int32)
            bb = jnp.full((L,), b, jnp.int32)
            u0 = plsc.load_gather(ublk, [ss, bb, c_lo, ulane])
            u1 = plsc.load_gather(ublk, [ss, bb, c_hi, ulane])
            v0 = plsc.load_gather(iblk, [ss, bb, c_lo, vlane])
            v1 = plsc.load_gather(iblk, [ss, bb, c_hi, vlane])
            psum[e0 + j, pl.ds(0, L)] = u0 * v0 + u1 * v1

    fire_batch(load_round_vecs(0), 0, 0)

    def round_body(t, carry):
        e0 = t * L
        vecs = load_round_vecs(t)
        for k in range(BPR):
            cur = k % 2
            nxt = 1 - cur
            if k + 1 < BPR:
                fire_batch(vecs, k + 1, nxt)
            else:
                @pl.when(t < ROUNDS - 1)
                def _():
                    fire_batch(load_round_vecs(t + 1), 0, nxt)

            drain(cur)
            extract_batch(e0, vecs, k, cur)
        return carry

    lax.fori_loop(0, ROUNDS, round_body, 0)

    pltpu.make_async_copy(ub_hbm.at[pl.ds(0, BPW)], ubv, sem_ubias).wait()
    pltpu.make_async_copy(ib_hbm.at[pl.ds(0, BPW)], ibv, sem_ibias).wait()

    def group_body(g, carry):
        off = g * L
        rows = iota + off
        acc = ubv[pl.ds(off, L)] + ibv[pl.ds(off, L)]
        for j in range(L):
            col = jnp.full((L,), j, jnp.int32)
            acc = acc + plsc.load_gather(psum, [rows, col])
        res[pl.ds(off, L)] = acc
        return carry

    lax.fori_loop(0, ROUNDS, group_body, 0)

    pltpu.sync_copy(res, out_hbm.at[pl.ds(base, BPW)])


def kernel(users, items, user_table, item_table, user_bias, item_bias):
    return _mf_sc(users, items, user_table.T, item_table.T,
                  user_bias, item_bias)
